# baseline (speedup 1.0000x reference)
import jax
import jax.numpy as jnp
from jax import lax
from jax.experimental import pallas as pl
from jax.experimental.pallas import tpu as pltpu
from jax.experimental.pallas import tpu_sc as plsc

B = 2
N = 10000
C = 128
K = 16
N_TARGET = 10
NEG_SLOPE = 0.01
N_PAD = 10240
T = 16
EPT = N_PAD // T
CHUNK = 128
NCH = EPT // CHUNK
LANES = 16
CW = 128
CS = 16


def _build_indices(nn_idx):
    perm = jax.random.permutation(jax.random.key(42), 2 * K - 1)
    cols = perm[: K - 1]
    sample = jnp.take(nn_idx, cols, axis=2)
    self_idx = jnp.broadcast_to(
        jnp.arange(N, dtype=jnp.int32)[None, :, None], (B, N, 1))
    nidx = jnp.concatenate([self_idx, sample], axis=2)
    pad = jnp.broadcast_to(
        jnp.arange(N, N_PAD, dtype=jnp.int32)[None, :, None],
        (B, N_PAD - N, K))
    nidx = jnp.concatenate([nidx, pad], axis=1)

    return nidx.reshape(B, T, NCH, CHUNK, K).transpose(0, 1, 2, 4, 3)


def _sc_agg(y3, idx_s):
    mesh = plsc.VectorSubcoreMesh(core_axis_name="c", subcore_axis_name="s")
    out_type = jax.ShapeDtypeStruct((B, N_PAD, C), jnp.float32)
    scratch = [
        pltpu.VMEM((K, CHUNK), jnp.int32),
        pltpu.VMEM((K, CHUNK), jnp.int32),
        pltpu.VMEM((CHUNK, C), jnp.float32),
        pltpu.VMEM((CHUNK, C), jnp.float32),
        pltpu.VMEM_SHARED((N_PAD, C), jnp.float32),
        pltpu.SemaphoreType.DMA,
        pltpu.SemaphoreType.DMA,
    ]

    def body(y_hbm, idxs_hbm, sums_hbm, idxs0, idxs1, acc0, acc1, out_sh,
             sem_g, sem_s):
        b = lax.axis_index("c")
        t = lax.axis_index("s")
        base = t * EPT
        y_b = y_hbm.at[b]

        z16 = jnp.zeros((LANES,), jnp.float32)

        def zrow(i, _):
            for j in range(C // LANES):
                acc0[i, pl.ds(j * LANES, LANES)] = z16
            return 0

        lax.fori_loop(0, CHUNK, zrow, 0)
        for c in range(NCH):
            pltpu.sync_copy(acc0, out_sh.at[pl.ds(base + c * CHUNK, CHUNK)])
        plsc.subcore_barrier()

        accs = (acc0, acc1)
        iss = (idxs0, idxs1)
        pend = [[], []]
        gh = [[], []]
        pltpu.sync_copy(idxs_hbm.at[b, t, 0], idxs0)
        pltpu.sync_copy(y_b.at[pl.ds(base, CHUNK)], acc0)
        gh[0] = [pltpu.async_copy(y_b.at[idxs0.at[k]], acc0, sem_g, add=True)
                 for k in range(1, K)]
        for c in range(NCH):
            p = c % 2
            q = (c + 1) % 2
            acc, isv = accs[p], iss[p]
            if c + 1 < NCH:
                for h in pend[q]:
                    h.wait()
                pend[q] = []
                pltpu.sync_copy(idxs_hbm.at[b, t, c + 1], iss[q])
                pltpu.sync_copy(
                    y_b.at[pl.ds(base + (c + 1) * CHUNK, CHUNK)], accs[q])
            for h in gh[p]:
                h.wait()
            if c + 1 < NCH:
                gh[q] = [pltpu.async_copy(y_b.at[iss[q].at[k]], accs[q],
                                          sem_g, add=True)
                         for k in range(1, K)]
            pend[p] = [pltpu.async_copy(acc, out_sh.at[isv.at[k]], sem_s,
                                        add=True) for k in range(K)]
        for p in (0, 1):
            for h in pend[p]:
                h.wait()
        plsc.subcore_barrier()

        pltpu.sync_copy(out_sh.at[pl.ds(base, EPT)],
                        sums_hbm.at[b, pl.ds(base, EPT)])

    fn = pl.kernel(body, out_type=out_type, mesh=mesh,
                   scratch_types=tuple(scratch))
    return fn(y3, idx_s)


def _sc_counts(idx_s):
    mesh = plsc.VectorSubcoreMesh(core_axis_name="c", subcore_axis_name="s")
    out_type = jax.ShapeDtypeStruct((B, N_PAD, CW), jnp.float32)
    scratch = [
        pltpu.VMEM((K, CHUNK), jnp.int32),
        pltpu.VMEM((K, CHUNK), jnp.int32),
        pltpu.VMEM((CHUNK, CW), jnp.float32),
        pltpu.VMEM((CHUNK, CW), jnp.float32),
        pltpu.VMEM_SHARED((N_PAD, CW), jnp.float32),
        pltpu.SemaphoreType.DMA,
    ]

    def body(idxs_hbm, counts_hbm, idxs0, idxs1, ones_v, zc_v, cnt_sh, sem):
        b = lax.axis_index("c")
        t = lax.axis_index("s")
        base = t * EPT
        o16 = jnp.full((LANES,), 1.0, jnp.float32)
        z16 = jnp.zeros((LANES,), jnp.float32)

        def orow(i, _):
            for j in range(CW // LANES):
                ones_v[i, pl.ds(j * LANES, LANES)] = o16
                zc_v[i, pl.ds(j * LANES, LANES)] = z16
            return 0

        lax.fori_loop(0, CHUNK, orow, 0)
        for c in range(NCH):
            pltpu.sync_copy(zc_v, cnt_sh.at[pl.ds(base + c * CHUNK, CHUNK)])
        plsc.subcore_barrier()
        iss = (idxs0, idxs1)
        pend = [[], []]
        for c in range(NCH):
            p = c % 2
            isv = iss[p]
            for h in pend[p]:
                h.wait()
            pltpu.sync_copy(idxs_hbm.at[b, t, c], isv)
            pend[p] = [pltpu.async_copy(ones_v, cnt_sh.at[isv.at[k]], sem,
                                        add=True) for k in range(1, K)]
        for p in (0, 1):
            for h in pend[p]:
                h.wait()
        plsc.subcore_barrier()
        pltpu.sync_copy(cnt_sh.at[pl.ds(base, EPT)],
                        counts_hbm.at[b, pl.ds(base, EPT)])

    fn = pl.kernel(body, out_type=out_type, mesh=mesh,
                   scratch_types=tuple(scratch))
    return fn(idx_s)


def _tc_mm0(x_flat, W, b2):
    R = 512
    G = x_flat.shape[0] // R

    def body(x_ref, w_ref, b_ref, o_ref):
        o_ref[...] = jnp.dot(x_ref[...], w_ref[...],
                             preferred_element_type=jnp.float32) + b_ref[...]

    return pl.pallas_call(
        body,
        grid=(G,),
        in_specs=[pl.BlockSpec((R, C), lambda i: (i, 0)),
                  pl.BlockSpec((C, C), lambda i: (0, 0)),
                  pl.BlockSpec((1, C), lambda i: (0, 0))],
        out_specs=pl.BlockSpec((R, C), lambda i: (i, 0)),
        out_shape=jax.ShapeDtypeStruct((x_flat.shape[0], C), jnp.float32),
    )(x_flat, W, b2)


def _tc_mm1(sums, counts, W, b2):
    R = 512
    G = sums.shape[0] // R

    def body(s_ref, c_ref, w_ref, b_ref, o_ref):
        cnt = c_ref[:, 0:1] + 1.0
        h = s_ref[...] / (16.0 * cnt)
        h = jnp.where(h >= 0, h, NEG_SLOPE * h)
        o_ref[...] = jnp.dot(h, w_ref[...],
                             preferred_element_type=jnp.float32) + b_ref[...]

    return pl.pallas_call(
        body,
        grid=(G,),
        in_specs=[pl.BlockSpec((R, C), lambda i: (i, 0)),
                  pl.BlockSpec((R, CS), lambda i: (i, 0)),
                  pl.BlockSpec((C, C), lambda i: (0, 0)),
                  pl.BlockSpec((1, C), lambda i: (0, 0))],
        out_specs=pl.BlockSpec((R, C), lambda i: (i, 0)),
        out_shape=jax.ShapeDtypeStruct((sums.shape[0], C), jnp.float32),
    )(sums, counts, W, b2)


def _tc_fin(sums, counts, W_fc, bfc2):
    def body(s_ref, c_ref, w_ref, b_ref, o_ref):
        rows = lax.broadcasted_iota(jnp.int32, (N_PAD, 1), 0)
        outs = []
        for b in range(B):
            s = s_ref[b]
            cnt = c_ref[b][:, 0:1] + 1.0
            h = s / (16.0 * cnt)
            h = jnp.where(h >= 0, h, NEG_SLOPE * h)
            h = jnp.where(rows < N, h, 0.0)
            p8 = h.reshape(N_PAD // 8, 8, C).sum(axis=0)
            r8 = jnp.dot(p8, w_ref[...], preferred_element_type=jnp.float32)
            outs.append(r8.sum(axis=0, keepdims=True) * (1.0 / N) + b_ref[...])
        o_ref[...] = jnp.concatenate(outs, axis=0)

    return pl.pallas_call(
        body,
        out_shape=jax.ShapeDtypeStruct((B, N_TARGET), jnp.float32),
    )(sums, counts, W_fc, bfc2)


def kernel(x, nn_idx, W0, b0, W1, b1, W_fc, b_fc):
    idx_s = _build_indices(nn_idx.astype(jnp.int32))
    xpad = jnp.pad(x, ((0, 0), (0, N_PAD - N), (0, 0))).reshape(B * N_PAD, C)
    y0 = _tc_mm0(xpad, W0, b0.reshape(1, C))
    counts = _sc_counts(idx_s)[..., :CS]
    sums0 = _sc_agg(y0.reshape(B, N_PAD, C), idx_s)
    y1 = _tc_mm1(sums0.reshape(B * N_PAD, C), counts.reshape(B * N_PAD, CS),
                 W1, b1.reshape(1, C))
    sums1 = _sc_agg(y1.reshape(B, N_PAD, C), idx_s)
    return _tc_fin(sums1, counts, W_fc, b_fc.reshape(1, N_TARGET))


# speedup vs baseline: 1.0052x; 1.0052x over previous
import jax
import jax.numpy as jnp
from jax import lax
from jax.experimental import pallas as pl
from jax.experimental.pallas import tpu as pltpu
from jax.experimental.pallas import tpu_sc as plsc

B = 2
N = 10000
C = 128
K = 16
N_TARGET = 10
NEG_SLOPE = 0.01
N_PAD = 10240
T = 16
EPT = N_PAD // T
CHUNK = 128
NCH = EPT // CHUNK
LANES = 16
CW = 128
CS = 16


def _build_indices(nn_idx):
    perm = jax.random.permutation(jax.random.key(42), 2 * K - 1)
    cols = perm[: K - 1]
    sample = jnp.take(nn_idx, cols, axis=2)
    self_idx = jnp.broadcast_to(
        jnp.arange(N, dtype=jnp.int32)[None, :, None], (B, N, 1))
    nidx = jnp.concatenate([self_idx, sample], axis=2)
    pad = jnp.broadcast_to(
        jnp.arange(N, N_PAD, dtype=jnp.int32)[None, :, None],
        (B, N_PAD - N, K))
    nidx = jnp.concatenate([nidx, pad], axis=1)

    return nidx.reshape(B, T, NCH, CHUNK, K).transpose(0, 1, 2, 4, 3)


def _sc_agg(y3, idx_s):
    mesh = plsc.VectorSubcoreMesh(core_axis_name="c", subcore_axis_name="s")
    out_type = jax.ShapeDtypeStruct((B, N_PAD, C), jnp.float32)
    scratch = [
        pltpu.VMEM((K, CHUNK), jnp.int32),
        pltpu.VMEM((K, CHUNK), jnp.int32),
        pltpu.VMEM((CHUNK, C), jnp.float32),
        pltpu.VMEM((CHUNK, C), jnp.float32),
        pltpu.VMEM_SHARED((N_PAD, C), jnp.float32),
        pltpu.SemaphoreType.DMA,
        pltpu.SemaphoreType.DMA,
    ]

    def body(y_hbm, idxs_hbm, sums_hbm, idxs0, idxs1, acc0, acc1, out_sh,
             sem_g, sem_s):
        b = lax.axis_index("c")
        t = lax.axis_index("s")
        base = t * EPT
        y_b = y_hbm.at[b]

        z16 = jnp.zeros((LANES,), jnp.float32)

        def zrow(i, _):
            for j in range(C // LANES):
                acc0[i, pl.ds(j * LANES, LANES)] = z16
            return 0

        lax.fori_loop(0, CHUNK, zrow, 0)
        for c in range(NCH):
            pltpu.sync_copy(acc0, out_sh.at[pl.ds(base + c * CHUNK, CHUNK)])
        plsc.subcore_barrier()

        accs = (acc0, acc1)
        iss = (idxs0, idxs1)
        pend = [[], []]
        gh = [[], []]
        pltpu.sync_copy(idxs_hbm.at[b, t, 0], idxs0)
        pltpu.sync_copy(y_b.at[pl.ds(base, CHUNK)], acc0)
        gh[0] = [pltpu.async_copy(y_b.at[idxs0.at[k]], acc0, sem_g, add=True)
                 for k in range(1, K)]
        for c in range(NCH):
            p = c % 2
            q = (c + 1) % 2
            acc, isv = accs[p], iss[p]
            if c + 1 < NCH:
                for h in pend[q]:
                    h.wait()
                pend[q] = []
                pltpu.sync_copy(idxs_hbm.at[b, t, c + 1], iss[q])
                pltpu.sync_copy(
                    y_b.at[pl.ds(base + (c + 1) * CHUNK, CHUNK)], accs[q])
            for h in gh[p]:
                h.wait()
            if c + 1 < NCH:
                gh[q] = [pltpu.async_copy(y_b.at[iss[q].at[k]], accs[q],
                                          sem_g, add=True)
                         for k in range(1, K)]
            pend[p] = [pltpu.async_copy(acc, out_sh.at[isv.at[k]], sem_s,
                                        add=True) for k in range(K)]
        for p in (0, 1):
            for h in pend[p]:
                h.wait()
        plsc.subcore_barrier()

        pltpu.sync_copy(out_sh.at[pl.ds(base, EPT)],
                        sums_hbm.at[b, pl.ds(base, EPT)])

    fn = pl.kernel(body, out_type=out_type, mesh=mesh,
                   scratch_types=tuple(scratch))
    return fn(y3, idx_s)


def _sc_counts(idx_s):
    mesh = plsc.VectorSubcoreMesh(core_axis_name="c", subcore_axis_name="s")
    out_type = jax.ShapeDtypeStruct((B, N_PAD, CW), jnp.float32)
    scratch = [
        pltpu.VMEM((K, CHUNK), jnp.int32),
        pltpu.VMEM((K, CHUNK), jnp.int32),
        pltpu.VMEM((CHUNK, CW), jnp.float32),
        pltpu.VMEM((CHUNK, CW), jnp.float32),
        pltpu.VMEM_SHARED((N_PAD, CW), jnp.float32),
        pltpu.SemaphoreType.DMA,
    ]

    def body(idxs_hbm, counts_hbm, idxs0, idxs1, ones_v, zc_v, cnt_sh, sem):
        b = lax.axis_index("c")
        t = lax.axis_index("s")
        base = t * EPT
        o16 = jnp.full((LANES,), 1.0, jnp.float32)
        z16 = jnp.zeros((LANES,), jnp.float32)

        def orow(i, _):
            for j in range(CW // LANES):
                ones_v[i, pl.ds(j * LANES, LANES)] = o16
                zc_v[i, pl.ds(j * LANES, LANES)] = z16
            return 0

        lax.fori_loop(0, CHUNK, orow, 0)
        for c in range(NCH):
            pltpu.sync_copy(zc_v, cnt_sh.at[pl.ds(base + c * CHUNK, CHUNK)])
        plsc.subcore_barrier()
        iss = (idxs0, idxs1)
        pend = [[], []]
        for c in range(NCH):
            p = c % 2
            isv = iss[p]
            for h in pend[p]:
                h.wait()
            pltpu.sync_copy(idxs_hbm.at[b, t, c], isv)
            pend[p] = [pltpu.async_copy(ones_v, cnt_sh.at[isv.at[k]], sem,
                                        add=True) for k in range(1, K)]
        for p in (0, 1):
            for h in pend[p]:
                h.wait()
        plsc.subcore_barrier()
        pltpu.sync_copy(cnt_sh.at[pl.ds(base, EPT)],
                        counts_hbm.at[b, pl.ds(base, EPT)])

    fn = pl.kernel(body, out_type=out_type, mesh=mesh,
                   scratch_types=tuple(scratch))
    return fn(idx_s)


def _tc_mm0(x, W, b2):
    R = 400
    G = N // R + 1

    def body(x_ref, w_ref, b_ref, o_ref):
        o_ref[0] = jnp.dot(x_ref[0], w_ref[...],
                           preferred_element_type=jnp.float32) + b_ref[...]

    return pl.pallas_call(
        body,
        grid=(B, G),
        in_specs=[pl.BlockSpec((1, R, C),
                               lambda b, i: (b, jnp.minimum(i, N // R - 1), 0)),
                  pl.BlockSpec((C, C), lambda b, i: (0, 0)),
                  pl.BlockSpec((1, C), lambda b, i: (0, 0))],
        out_specs=pl.BlockSpec((1, R, C), lambda b, i: (b, i, 0)),
        out_shape=jax.ShapeDtypeStruct((B, N_PAD, C), jnp.float32),
    )(x, W, b2)


def _tc_mm1(sums, counts, W, b2):
    R = 512
    G = sums.shape[0] // R

    def body(s_ref, c_ref, w_ref, b_ref, o_ref):
        cnt = c_ref[:, 0:1] + 1.0
        h = s_ref[...] / (16.0 * cnt)
        h = jnp.where(h >= 0, h, NEG_SLOPE * h)
        o_ref[...] = jnp.dot(h, w_ref[...],
                             preferred_element_type=jnp.float32) + b_ref[...]

    return pl.pallas_call(
        body,
        grid=(G,),
        in_specs=[pl.BlockSpec((R, C), lambda i: (i, 0)),
                  pl.BlockSpec((R, CS), lambda i: (i, 0)),
                  pl.BlockSpec((C, C), lambda i: (0, 0)),
                  pl.BlockSpec((1, C), lambda i: (0, 0))],
        out_specs=pl.BlockSpec((R, C), lambda i: (i, 0)),
        out_shape=jax.ShapeDtypeStruct((sums.shape[0], C), jnp.float32),
    )(sums, counts, W, b2)


def _tc_fin(sums, counts, W_fc, bfc2):
    def body(s_ref, c_ref, w_ref, b_ref, o_ref):
        rows = lax.broadcasted_iota(jnp.int32, (N_PAD, 1), 0)
        outs = []
        for b in range(B):
            s = s_ref[b]
            cnt = c_ref[b][:, 0:1] + 1.0
            h = s / (16.0 * cnt)
            h = jnp.where(h >= 0, h, NEG_SLOPE * h)
            h = jnp.where(rows < N, h, 0.0)
            p8 = h.reshape(N_PAD // 8, 8, C).sum(axis=0)
            r8 = jnp.dot(p8, w_ref[...], preferred_element_type=jnp.float32)
            outs.append(r8.sum(axis=0, keepdims=True) * (1.0 / N) + b_ref[...])
        o_ref[...] = jnp.concatenate(outs, axis=0)

    return pl.pallas_call(
        body,
        out_shape=jax.ShapeDtypeStruct((B, N_TARGET), jnp.float32),
    )(sums, counts, W_fc, bfc2)


def kernel(x, nn_idx, W0, b0, W1, b1, W_fc, b_fc):
    idx_s = _build_indices(nn_idx.astype(jnp.int32))
    y0 = _tc_mm0(x, W0, b0.reshape(1, C))
    counts = _sc_counts(idx_s)[..., :CS]
    sums0 = _sc_agg(y0, idx_s)
    y1 = _tc_mm1(sums0.reshape(B * N_PAD, C), counts.reshape(B * N_PAD, CS),
                 W1, b1.reshape(1, C))
    sums1 = _sc_agg(y1.reshape(B, N_PAD, C), idx_s)
    return _tc_fin(sums1, counts, W_fc, b_fc.reshape(1, N_TARGET))
